# baseline (speedup 1.0000x reference)
import functools

import jax
import jax.numpy as jnp
from jax import lax
from jax.experimental import pallas as pl
from jax.experimental.pallas import tpu as pltpu
from jax.experimental.pallas import tpu_sc as plsc

_LANES = 16


def _sc_body(rows_per_w, chunk_rows, D, B, S,
             x_hbm, pos_hbm, out_hbm, pos_v, x_v, o_v):
    wid = lax.axis_index("s") * 2 + lax.axis_index("c")
    base = wid * rows_per_w * D
    n_chunks = rows_per_w // chunk_rows
    chunk = chunk_rows * D
    vec_iters = chunk // _LANES

    def add_loop(i, _):
        sl = pl.ds(i * _LANES, _LANES)
        o_v[sl] = x_v[sl] + pos_v[sl]
        return ()

    def chunk_body(c, _):
        off = base + c * chunk
        pltpu.sync_copy(pos_hbm.at[pl.ds(off, chunk)], pos_v)
        for b in range(B):
            pltpu.sync_copy(x_hbm.at[pl.ds(b * S * D + off, chunk)], x_v)
            lax.fori_loop(0, vec_iters, add_loop, (), unroll=8)
            pltpu.sync_copy(o_v, out_hbm.at[pl.ds(b * S * D + off, chunk)])
        return ()

    lax.fori_loop(0, n_chunks, chunk_body, ())


def _sc_add(x1, pos1, B, S, D):
    info = plsc.get_sparse_core_info()
    nw = info.num_cores * info.num_subcores
    rows_per_w = S // nw
    chunk_rows = min(16, rows_per_w)
    mesh = plsc.VectorSubcoreMesh(core_axis_name="c", subcore_axis_name="s")
    buf = chunk_rows * D
    f = pl.kernel(
        functools.partial(_sc_body, rows_per_w, chunk_rows, D, B, S),
        mesh=mesh,
        out_type=jax.ShapeDtypeStruct((B * S * D,), jnp.float32),
        scratch_types=[
            pltpu.VMEM((buf,), jnp.float32),
            pltpu.VMEM((buf,), jnp.float32),
            pltpu.VMEM((buf,), jnp.float32),
        ],
    )
    return f(x1, pos1)


def kernel(x, pos_table):
    B, S, D = x.shape
    assert S <= pos_table.shape[0] and S % 32 == 0
    x1 = x.reshape(-1)
    pos1 = pos_table[:S].reshape(-1)
    out = _sc_add(x1, pos1, B, S, D)
    return out.reshape(B, S, D)


# speedup vs baseline: 1.2471x; 1.2471x over previous
import functools

import jax
import jax.numpy as jnp
from jax import lax
from jax.experimental import pallas as pl
from jax.experimental.pallas import tpu as pltpu
from jax.experimental.pallas import tpu_sc as plsc

_LANES = 16


def _sc_body(rows_per_w, chunk_rows, D, B, S, n_chunks,
             x_hbm, pos_hbm, out_hbm,
             pos_v0, pos_v1, x_v0, x_v1, o_v0, o_v1,
             sp0, sp1, sx0, sx1, so0, so1):
    pos_v = (pos_v0, pos_v1)
    x_v = (x_v0, x_v1)
    o_v = (o_v0, o_v1)
    sp = (sp0, sp1)
    sx = (sx0, sx1)
    so = (so0, so1)

    wid = lax.axis_index("s") * 2 + lax.axis_index("c")
    chunk = chunk_rows * D
    base = wid * rows_per_w * D
    vec_iters = chunk // _LANES
    cmask = n_chunks - 1

    def start_pos(c, p):
        pltpu.make_async_copy(
            pos_hbm.at[pl.ds(base + c * chunk, chunk)], pos_v[p], sp[p]).start()

    def wait_pos(p):
        pltpu.make_async_copy(
            pos_hbm.at[pl.ds(0, chunk)], pos_v[p], sp[p]).wait()

    def start_x(c, b, p):
        pltpu.make_async_copy(
            x_hbm.at[pl.ds(b * S * D + base + c * chunk, chunk)],
            x_v[p], sx[p]).start()

    def wait_x(p):
        pltpu.make_async_copy(
            x_hbm.at[pl.ds(0, chunk)], x_v[p], sx[p]).wait()

    def start_out(c, b, p):
        pltpu.make_async_copy(
            o_v[p], out_hbm.at[pl.ds(b * S * D + base + c * chunk, chunk)],
            so[p]).start()

    def wait_out(p):
        pltpu.make_async_copy(
            o_v[p], out_hbm.at[pl.ds(0, chunk)], so[p]).wait()

    def item(c, b, par, first):
        xb = b % 2
        wait_x(xb)
        if b == B - 1:
            start_x((c + 1) & cmask, 0, 0)
        else:
            start_x(c, b + 1, 1 - xb)
        if not first:
            wait_out(b % 2)

        ob = o_v[b % 2]
        xv = x_v[xb]
        pv = pos_v[par]

        def add_loop(i, _):
            sl = pl.ds(i * _LANES, _LANES)
            ob[sl] = xv[sl] + pv[sl]
            return ()

        lax.fori_loop(0, vec_iters, add_loop, (), unroll=8)
        start_out(c, b, b % 2)

    def do_chunk(c, par, first_pair):
        wait_pos(par)
        start_pos((c + 1) & cmask, 1 - par)
        for b in range(B):
            item(c, b, par, first=(first_pair and b < 2))

    start_pos(0, 0)
    start_x(0, 0, 0)
    do_chunk(0, 0, True)
    do_chunk(1, 1, False)

    def pair_body(c2, _):
        do_chunk(2 * c2, 0, False)
        do_chunk(2 * c2 + 1, 1, False)
        return ()

    lax.fori_loop(1, n_chunks // 2, pair_body, ())

    wait_pos(0)
    wait_x(0)
    wait_out(0)
    wait_out(1)


def _sc_add(x1, pos1, B, S, D):
    info = plsc.get_sparse_core_info()
    nw = info.num_cores * info.num_subcores
    rows_per_w = S // nw
    chunk_rows = min(16, rows_per_w)
    n_chunks = rows_per_w // chunk_rows
    assert n_chunks >= 2 and n_chunks % 2 == 0 and (n_chunks & (n_chunks - 1)) == 0
    mesh = plsc.VectorSubcoreMesh(core_axis_name="c", subcore_axis_name="s")
    buf = chunk_rows * D
    f = pl.kernel(
        functools.partial(_sc_body, rows_per_w, chunk_rows, D, B, S, n_chunks),
        mesh=mesh,
        out_type=jax.ShapeDtypeStruct((B * S * D,), jnp.float32),
        scratch_types=(
            [pltpu.VMEM((buf,), jnp.float32)] * 6
            + [pltpu.SemaphoreType.DMA] * 6
        ),
    )
    return f(x1, pos1)


def kernel(x, pos_table):
    B, S, D = x.shape
    assert S <= pos_table.shape[0] and S % 32 == 0
    x1 = x.reshape(-1)
    pos1 = pos_table[:S].reshape(-1)
    out = _sc_add(x1, pos1, B, S, D)
    return out.reshape(B, S, D)


# speedup vs baseline: 5.6757x; 4.5511x over previous
import functools

import jax
import jax.numpy as jnp
from jax import lax
from jax.experimental import pallas as pl
from jax.experimental.pallas import tpu as pltpu
from jax.experimental.pallas import tpu_sc as plsc

_LANES = 16


def _sc_body(rows_per_w, chunk_rows, D, B, n_chunks,
             x_hbm, pos_hbm, out_hbm,
             pos_v0, pos_v1, x_v0, x_v1, o_v0, o_v1,
             sp0, sp1, sx0, sx1, so0, so1):
    pos_v = (pos_v0, pos_v1)
    x_v = (x_v0, x_v1)
    o_v = (o_v0, o_v1)
    sp = (sp0, sp1)
    sx = (sx0, sx1)
    so = (so0, so1)

    wid = lax.axis_index("s") * 2 + lax.axis_index("c")
    row0 = wid * rows_per_w
    vec_iters = (chunk_rows * D) // _LANES
    row_iters = D // _LANES
    rsh = row_iters.bit_length() - 1
    jmask = row_iters - 1
    cmask = n_chunks - 1

    def start_pos(c, p):
        pltpu.make_async_copy(
            pos_hbm.at[pl.ds(row0 + c * chunk_rows, chunk_rows), :],
            pos_v[p], sp[p]).start()

    def wait_pos(p):
        pltpu.make_async_copy(
            pos_hbm.at[pl.ds(0, chunk_rows), :], pos_v[p], sp[p]).wait()

    def start_x(c, b, p):
        pltpu.make_async_copy(
            x_hbm.at[b, pl.ds(row0 + c * chunk_rows, chunk_rows), :],
            x_v[p], sx[p]).start()

    def wait_x(p):
        pltpu.make_async_copy(
            x_hbm.at[0, pl.ds(0, chunk_rows), :], x_v[p], sx[p]).wait()

    def start_out(c, b, p):
        pltpu.make_async_copy(
            o_v[p], out_hbm.at[b, pl.ds(row0 + c * chunk_rows, chunk_rows), :],
            so[p]).start()

    def wait_out(p):
        pltpu.make_async_copy(
            o_v[p], out_hbm.at[0, pl.ds(0, chunk_rows), :], so[p]).wait()

    def item(c, b, par, first):
        xb = b % 2
        wait_x(xb)
        if b == B - 1:
            start_x((c + 1) & cmask, 0, 0)
        else:
            start_x(c, b + 1, 1 - xb)
        if not first:
            wait_out(b % 2)

        ob = o_v[b % 2]
        xv = x_v[xb]
        pv = pos_v[par]

        def add_loop(i, _):
            r = i >> rsh
            sl = pl.ds((i & jmask) * _LANES, _LANES)
            ob[r, sl] = xv[r, sl] + pv[r, sl]
            return ()

        lax.fori_loop(0, vec_iters, add_loop, (), unroll=8)
        start_out(c, b, b % 2)

    def do_chunk(c, par, first_pair):
        wait_pos(par)
        start_pos((c + 1) & cmask, 1 - par)
        for b in range(B):
            item(c, b, par, first=(first_pair and b < 2))

    start_pos(0, 0)
    start_x(0, 0, 0)
    do_chunk(0, 0, True)
    do_chunk(1, 1, False)

    def pair_body(c2, _):
        do_chunk(2 * c2, 0, False)
        do_chunk(2 * c2 + 1, 1, False)
        return ()

    lax.fori_loop(1, n_chunks // 2, pair_body, ())

    wait_pos(0)
    wait_x(0)
    wait_out(0)
    wait_out(1)


def _sc_add(x, pos, B, S, D):
    info = plsc.get_sparse_core_info()
    nw = info.num_cores * info.num_subcores
    rows_per_w = S // nw
    chunk_rows = min(16, rows_per_w)
    n_chunks = rows_per_w // chunk_rows
    assert n_chunks >= 2 and (n_chunks & (n_chunks - 1)) == 0
    mesh = plsc.VectorSubcoreMesh(core_axis_name="c", subcore_axis_name="s")
    f = pl.kernel(
        functools.partial(_sc_body, rows_per_w, chunk_rows, D, B, n_chunks),
        mesh=mesh,
        out_type=jax.ShapeDtypeStruct((B, S, D), jnp.float32),
        scratch_types=(
            [pltpu.VMEM((chunk_rows, D), jnp.float32)] * 6
            + [pltpu.SemaphoreType.DMA] * 6
        ),
    )
    return f(x, pos)


def kernel(x, pos_table):
    B, S, D = x.shape
    assert S <= pos_table.shape[0] and S % 32 == 0
    return _sc_add(x, pos_table[:S], B, S, D)
